# baseline (speedup 1.0000x reference)
import jax
import jax.numpy as jnp
from jax.experimental import pallas as pl
from jax.experimental.pallas import tpu as pltpu

GRID_H = 32
GRID_W = 32
N = GRID_H * GRID_W
C = 10
SIG2 = 2.0

BM = 1024


def _som_kernel(x_ref, w_ref, labels_ref, out_ref, wb_ref, tab_ref, w2_ref):
    def batch_step():
        xb = x_ref[...].astype(jnp.bfloat16)
        H = N // 2
        s0 = jax.lax.dot_general(
            xb, wb_ref[:H, :], (((1,), (1,)), ((), ())),
            preferred_element_type=jnp.float32)
        s1 = jax.lax.dot_general(
            xb, wb_ref[H:, :], (((1,), (1,)), ((), ())),
            preferred_element_type=jnp.float32)
        d0 = w2_ref[0:1, :H] - 2.0 * s0
        d1 = w2_ref[0:1, H:] - 2.0 * s1
        v0 = jnp.min(d0, axis=1)
        i0 = jnp.argmin(d0, axis=1)
        v1 = jnp.min(d1, axis=1)
        i1 = jnp.argmin(d1, axis=1) + H
        bmu = jnp.where(v1 < v0, i1, i0)
        onehot = (jax.lax.broadcasted_iota(jnp.int32, (BM, N), 1)
                  == bmu[:, None]).astype(jnp.float32)
        out_ref[...] = jax.lax.dot_general(
            onehot, tab_ref[...], (((1,), (0,)), ((), ())),
            preferred_element_type=jnp.float32)

    @pl.when(pl.program_id(0) == 0)
    def _():
        w = w_ref[...]
        wb_ref[...] = w.astype(jnp.bfloat16)
        w2_ref[...] = jnp.sum(w * w, axis=1, keepdims=True).reshape(1, N)
        n = jax.lax.broadcasted_iota(jnp.int32, (N, N), 0)
        m = jax.lax.broadcasted_iota(jnp.int32, (N, N), 1)
        di = ((n >> 5) - (m >> 5)).astype(jnp.float32)
        dj = ((n & 31) - (m & 31)).astype(jnp.float32)
        g = jnp.exp(-(di * di + dj * dj) / SIG2)
        tab = jax.lax.dot_general(
            g, labels_ref[...].reshape(N, C), (((1,), (0,)), ((), ())),
            preferred_element_type=jnp.float32)
        tab_ref[...] = tab / jnp.sum(g, axis=1, keepdims=True)
        batch_step()

    @pl.when(pl.program_id(0) > 0)
    def _():
        batch_step()


def kernel(x, som_weights, labels):
    B, D = x.shape
    out = pl.pallas_call(
        _som_kernel,
        grid=(B // BM,),
        in_specs=[
            pl.BlockSpec((BM, D), lambda i: (i, 0)),
            pl.BlockSpec((N, D), lambda i: (0, 0)),
            pl.BlockSpec((GRID_H, GRID_W, C), lambda i: (0, 0, 0)),
        ],
        out_specs=pl.BlockSpec((BM, C), lambda i: (i, 0)),
        out_shape=jax.ShapeDtypeStruct((B, C), jnp.float32),
        scratch_shapes=[
            pltpu.VMEM((N, D), jnp.bfloat16),
            pltpu.VMEM((N, C), jnp.float32),
            pltpu.VMEM((1, N), jnp.float32),
        ],
    )(x, som_weights, labels)
    return out


# speedup vs baseline: 1.2005x; 1.2005x over previous
import jax
import jax.numpy as jnp
from jax.experimental import pallas as pl
from jax.experimental.pallas import tpu as pltpu

GRID_H = 32
GRID_W = 32
N = GRID_H * GRID_W
C = 10
SIG2 = 2.0

BM = 1024


def _som_kernel(x_ref, w_ref, labels_ref, out_ref, wb_ref, tab_ref, w2_ref):
    def batch_step():
        s = jax.lax.dot_general(
            x_ref[...].astype(jnp.bfloat16), wb_ref[...],
            (((1,), (1,)), ((), ())),
            preferred_element_type=jnp.float32)
        d = w2_ref[...] - 2.0 * s
        bmu = jnp.argmin(d, axis=1)
        onehot = (jax.lax.broadcasted_iota(jnp.int32, (BM, N), 1)
                  == bmu[:, None]).astype(jnp.float32)
        out_ref[...] = jax.lax.dot_general(
            onehot, tab_ref[...], (((1,), (0,)), ((), ())),
            preferred_element_type=jnp.float32)

    @pl.when(pl.program_id(0) == 0)
    def _():
        w = w_ref[...]
        wb_ref[...] = w.astype(jnp.bfloat16)
        w2_ref[...] = jnp.sum(w * w, axis=1, keepdims=True).reshape(1, N)
        n = jax.lax.broadcasted_iota(jnp.int32, (N, N), 0)
        m = jax.lax.broadcasted_iota(jnp.int32, (N, N), 1)
        di = ((n >> 5) - (m >> 5)).astype(jnp.float32)
        dj = ((n & 31) - (m & 31)).astype(jnp.float32)
        g = jnp.exp(-(di * di + dj * dj) / SIG2)
        tab = jax.lax.dot_general(
            g, labels_ref[...].reshape(N, C), (((1,), (0,)), ((), ())),
            preferred_element_type=jnp.float32)
        tab_ref[...] = tab / jnp.sum(g, axis=1, keepdims=True)
        batch_step()

    @pl.when(pl.program_id(0) > 0)
    def _():
        batch_step()


def kernel(x, som_weights, labels):
    B, D = x.shape
    out = pl.pallas_call(
        _som_kernel,
        grid=(B // BM,),
        in_specs=[
            pl.BlockSpec((BM, D), lambda i: (i, 0)),
            pl.BlockSpec((N, D), lambda i: (0, 0)),
            pl.BlockSpec((GRID_H, GRID_W, C), lambda i: (0, 0, 0)),
        ],
        out_specs=pl.BlockSpec((BM, C), lambda i: (i, 0)),
        out_shape=jax.ShapeDtypeStruct((B, C), jnp.float32),
        scratch_shapes=[
            pltpu.VMEM((N, D), jnp.bfloat16),
            pltpu.VMEM((N, C), jnp.float32),
            pltpu.VMEM((1, N), jnp.float32),
        ],
    )(x, som_weights, labels)
    return out
